# baseline (speedup 1.0000x reference)
import functools

import jax
import jax.numpy as jnp
from jax import lax
from jax.experimental import pallas as pl
from jax.experimental.pallas import tpu as pltpu
from jax.experimental.pallas import tpu_sc as plsc

N = 10000
E = 320000
H = 128
DOUT = 64
G = 64

NC = 2
NS = 16
NW = NC * NS
CHUNK = 128
IB = 20
NBLK = 4
NCH = IB * NBLK
EPW = NCH * CHUNK
EPAD = NW * EPW
CH2 = 120
SIB = 12
NBLKS = 14
DEPTH = 3
NGRP = SIB // DEPTH
WHO = (0,) * 10 + (1,) * 4
EPW2 = NBLKS * SIB * CH2
EPADS = NS * EPW2
NPAD = 10112
RPS = NPAD // NS

RB = 1000
NB = N // RB


_MESH = plsc.VectorSubcoreMesh(core_axis_name="c", subcore_axis_name="s")


@functools.partial(
    pl.kernel,
    out_type=jax.ShapeDtypeStruct((NC, NPAD, H), jnp.float32),
    mesh=_MESH,
    scratch_types=[
        pltpu.VMEM((NCH, CHUNK), jnp.int32),
        pltpu.VMEM((CHUNK, H), jnp.float32),
        pltpu.VMEM_SHARED((NPAD, H), jnp.float32),
    ],
)
def _sc_deg(dst_hbm, degp_hbm, idxd, ones_v, deg_sh):
    c = lax.axis_index("c")
    s = lax.axis_index("s")
    wid = s * NC + c
    pltpu.sync_copy(dst_hbm.at[wid], idxd)
    zero16 = jnp.zeros((16,), jnp.float32)
    one16 = jnp.ones((16,), jnp.float32)

    def zbody(r, carry):
        for k2 in range(H // 16):
            ones_v[r, pl.ds(k2 * 16, 16)] = zero16
        return carry

    lax.fori_loop(0, CHUNK, zbody, 0)
    base = s * RPS
    off = 0
    for size in (128, 128, 128, 128, RPS - 512):
        pltpu.sync_copy(ones_v.at[pl.ds(0, size)],
                        deg_sh.at[pl.ds(base + off, size)])
        off += size

    def obody(r, carry):
        ones_v[r, pl.ds(0, 16)] = one16
        return carry

    lax.fori_loop(0, CHUNK, obody, 0)
    plsc.subcore_barrier()

    def body(j, carry):
        pltpu.sync_copy(ones_v, deg_sh.at[idxd.at[j]], add=True)
        return carry

    lax.fori_loop(0, NCH, body, 0)
    plsc.subcore_barrier()
    pltpu.sync_copy(deg_sh.at[pl.ds(base, RPS)],
                    degp_hbm.at[c, pl.ds(base, RPS)])


@functools.partial(
    pl.kernel,
    out_type=jax.ShapeDtypeStruct((NC, NPAD, H), jnp.float32),
    mesh=_MESH,
    scratch_types=[
        pltpu.VMEM((2 * SIB, CH2), jnp.int32),
        pltpu.VMEM((CH2, H), jnp.float32),
        pltpu.VMEM((CH2, H), jnp.float32),
        pltpu.VMEM((CH2, H), jnp.float32),
        pltpu.VMEM_SHARED((NPAD, H), jnp.float32),
        pltpu.SemaphoreType.DMA,
        pltpu.SemaphoreType.DMA,
        pltpu.SemaphoreType.DMA,
    ],
)
def _sc_spmm(m_hbm, idx_hbm, p_hbm, ib,
             r0, r1, r2, agg_sh, s0, s1, s2):
    c = lax.axis_index("c")
    s = lax.axis_index("s")
    rows = (r0, r1, r2)
    sems = (s0, s1, s2)
    zero16 = jnp.zeros((16,), jnp.float32)

    def zbody(r, carry):
        for k2 in range(H // 16):
            r0[r, pl.ds(k2 * 16, 16)] = zero16
        return carry

    lax.fori_loop(0, CH2, zbody, 0)
    base = s * RPS
    off = 0
    for size in (120,) * 5 + (RPS - 600,):
        pltpu.sync_copy(r0.at[pl.ds(0, size)],
                        agg_sh.at[pl.ds(base + off, size)])
        off += size
    plsc.subcore_barrier()

    for b in range(NBLKS):
        @pl.when(c == WHO[b])
        def _(b=b):
            pltpu.sync_copy(idx_hbm.at[s, b], ib)
            for k in range(DEPTH):
                pltpu.async_copy(m_hbm.at[ib.at[k]], rows[k], sems[k])

            def gbody(g, carry):
                for k in range(DEPTH):
                    j = g * DEPTH + k
                    pltpu.make_async_copy(m_hbm.at[ib.at[j]],
                                          rows[k], sems[k]).wait()
                    pltpu.sync_copy(rows[k], agg_sh.at[ib.at[SIB + j]],
                                    add=True)

                    @pl.when(g < NGRP - 1)
                    def _(k=k, j=j):
                        pltpu.async_copy(m_hbm.at[ib.at[j + DEPTH]],
                                         rows[k], sems[k])
                return carry

            lax.fori_loop(0, NGRP, gbody, 0)
    plsc.subcore_barrier()
    pltpu.sync_copy(agg_sh.at[pl.ds(base, RPS)],
                    p_hbm.at[c, pl.ds(base, RPS)])


def _tc1_body(x_ref, w_ref, degp_ref, m_ref, dinv_ref):
    deg = degp_ref[0, :, 0:1] + degp_ref[1, :, 0:1] + 1.0
    dinv = lax.rsqrt(deg)
    dinv_ref[...] = dinv
    m_ref[...] = jnp.dot(x_ref[...], w_ref[...],
                         preferred_element_type=jnp.float32) * dinv


_tc1 = pl.pallas_call(
    _tc1_body,
    grid=(NB,),
    in_specs=[
        pl.BlockSpec((RB, H), lambda i: (i, 0)),
        pl.BlockSpec((H, H), lambda i: (0, 0)),
        pl.BlockSpec((NC, RB, H), lambda i: (0, i, 0)),
    ],
    out_specs=[
        pl.BlockSpec((RB, H), lambda i: (i, 0)),
        pl.BlockSpec((RB, 1), lambda i: (i, 0)),
    ],
    out_shape=[
        jax.ShapeDtypeStruct((N, H), jnp.float32),
        jax.ShapeDtypeStruct((N, 1), jnp.float32),
    ],
)


def _tc_mid_body(p_ref, m_ref, dinv_ref, b_ref, w_ref, o_ref):
    dinv = dinv_ref[...]
    a = (p_ref[0] + p_ref[1] + m_ref[...]) * dinv + b_ref[...]
    a = jnp.maximum(a, 0.0)
    o_ref[...] = jnp.dot(a, w_ref[...],
                         preferred_element_type=jnp.float32) * dinv


_tc_mid = pl.pallas_call(
    _tc_mid_body,
    grid=(NB,),
    in_specs=[
        pl.BlockSpec((NC, RB, H), lambda i: (0, i, 0)),
        pl.BlockSpec((RB, H), lambda i: (i, 0)),
        pl.BlockSpec((RB, 1), lambda i: (i, 0)),
        pl.BlockSpec((1, H), lambda i: (0, 0)),
        pl.BlockSpec((H, H), lambda i: (0, 0)),
    ],
    out_specs=pl.BlockSpec((RB, H), lambda i: (i, 0)),
    out_shape=jax.ShapeDtypeStruct((N, H), jnp.float32),
)


def _tc4_body(p_ref, m_ref, dinv_ref, b_ref, batch_ref, wl_ref, bl_ref,
              o_ref, sums_ref, cnt_ref):
    i = pl.program_id(0)

    @pl.when(i == 0)
    def _():
        sums_ref[...] = jnp.zeros_like(sums_ref)
        cnt_ref[...] = jnp.zeros_like(cnt_ref)

    h = (p_ref[0] + p_ref[1] + m_ref[...]) * dinv_ref[...] + b_ref[...]
    onehot = (batch_ref[...] ==
              lax.broadcasted_iota(jnp.int32, (RB, G), 1)).astype(jnp.float32)
    sums_ref[...] += lax.dot_general(
        onehot, h, (((0,), (0,)), ((), ())),
        preferred_element_type=jnp.float32)
    cnt_ref[...] += lax.dot_general(
        onehot, jnp.ones((RB, H), jnp.float32), (((0,), (0,)), ((), ())),
        preferred_element_type=jnp.float32)

    @pl.when(i == NB - 1)
    def _():
        pooled = sums_ref[...] / jnp.maximum(cnt_ref[...], 1.0)
        o_ref[...] = jnp.dot(pooled, wl_ref[...],
                             preferred_element_type=jnp.float32) + bl_ref[...]


_tc4 = pl.pallas_call(
    _tc4_body,
    grid=(NB,),
    in_specs=[
        pl.BlockSpec((NC, RB, H), lambda i: (0, i, 0)),
        pl.BlockSpec((RB, H), lambda i: (i, 0)),
        pl.BlockSpec((RB, 1), lambda i: (i, 0)),
        pl.BlockSpec((1, H), lambda i: (0, 0)),
        pl.BlockSpec((RB, 1), lambda i: (i, 0)),
        pl.BlockSpec((H, DOUT), lambda i: (0, 0)),
        pl.BlockSpec((1, DOUT), lambda i: (0, 0)),
    ],
    out_specs=pl.BlockSpec((G, DOUT), lambda i: (0, 0)),
    out_shape=jax.ShapeDtypeStruct((G, DOUT), jnp.float32),
    scratch_shapes=[
        pltpu.VMEM((G, H), jnp.float32),
        pltpu.VMEM((G, H), jnp.float32),
    ],
)


def kernel(x, edge_index, batch, W1, b1, W2, b2, W3, b3, Wl, bl):
    pad = EPAD - E
    dst_p = jnp.concatenate([edge_index[1], jnp.full((pad,), N, jnp.int32)])
    dst3 = dst_p.reshape(NW, NCH, CHUNK)
    pad2 = EPADS - E
    src_p2 = jnp.concatenate([edge_index[0], jnp.zeros((pad2,), jnp.int32)])
    dst_p2 = jnp.concatenate([edge_index[1], jnp.full((pad2,), N, jnp.int32)])
    src4 = src_p2.reshape(NS, NBLKS, SIB, CH2)
    dst4 = dst_p2.reshape(NS, NBLKS, SIB, CH2)
    idx_comb = jnp.concatenate([src4, dst4], axis=2).reshape(
        NS, NBLKS, 2 * SIB, CH2)
    batch2 = batch.reshape(N, 1)

    degp = _sc_deg(dst3)
    m1, dinv = _tc1(x, W1, degp)
    p1 = _sc_spmm(m1, idx_comb)
    m2 = _tc_mid(p1, m1, dinv, b1.reshape(1, H), W2)
    p2 = _sc_spmm(m2, idx_comb)
    m3 = _tc_mid(p2, m2, dinv, b2.reshape(1, H), W3)
    p3 = _sc_spmm(m3, idx_comb)
    return _tc4(p3, m3, dinv, b3.reshape(1, H), batch2, Wl,
                bl.reshape(1, DOUT))


# speedup vs baseline: 1.0264x; 1.0264x over previous
import functools

import jax
import jax.numpy as jnp
from jax import lax
from jax.experimental import pallas as pl
from jax.experimental.pallas import tpu as pltpu
from jax.experimental.pallas import tpu_sc as plsc

N = 10000
E = 320000
H = 128
DOUT = 64
G = 64

NC = 2
NS = 16
NW = NC * NS
CHUNK = 128
IB = 20
NBLK = 4
NCH = IB * NBLK
EPW = NCH * CHUNK
EPAD = NW * EPW
CH2 = 112
SIB = 15
NBLKS = 12
DEPTH = 3
NGRP = SIB // DEPTH
WHO = (0,) * 9 + (1,) * 3
EPW2 = NBLKS * SIB * CH2
EPADS = NS * EPW2
NPAD = 10112
RPS = NPAD // NS

RB = 1000
NB = N // RB


_MESH = plsc.VectorSubcoreMesh(core_axis_name="c", subcore_axis_name="s")


@functools.partial(
    pl.kernel,
    out_type=jax.ShapeDtypeStruct((NC, NPAD, H), jnp.float32),
    mesh=_MESH,
    scratch_types=[
        pltpu.VMEM((NCH, CHUNK), jnp.int32),
        pltpu.VMEM((CHUNK, H), jnp.float32),
        pltpu.VMEM_SHARED((NPAD, H), jnp.float32),
    ],
)
def _sc_deg(dst_hbm, degp_hbm, idxd, ones_v, deg_sh):
    c = lax.axis_index("c")
    s = lax.axis_index("s")
    wid = s * NC + c
    pltpu.sync_copy(dst_hbm.at[wid], idxd)
    zero16 = jnp.zeros((16,), jnp.float32)
    one16 = jnp.ones((16,), jnp.float32)

    def zbody(r, carry):
        for k2 in range(H // 16):
            ones_v[r, pl.ds(k2 * 16, 16)] = zero16
        return carry

    lax.fori_loop(0, CHUNK, zbody, 0)
    base = s * RPS
    off = 0
    for size in (128, 128, 128, 128, RPS - 512):
        pltpu.sync_copy(ones_v.at[pl.ds(0, size)],
                        deg_sh.at[pl.ds(base + off, size)])
        off += size

    def obody(r, carry):
        ones_v[r, pl.ds(0, 16)] = one16
        return carry

    lax.fori_loop(0, CHUNK, obody, 0)
    plsc.subcore_barrier()

    def body(j, carry):
        pltpu.sync_copy(ones_v, deg_sh.at[idxd.at[j]], add=True)
        return carry

    lax.fori_loop(0, NCH, body, 0)
    plsc.subcore_barrier()
    pltpu.sync_copy(deg_sh.at[pl.ds(base, RPS)],
                    degp_hbm.at[c, pl.ds(base, RPS)])


@functools.partial(
    pl.kernel,
    out_type=jax.ShapeDtypeStruct((NC, NPAD, H), jnp.float32),
    mesh=_MESH,
    scratch_types=[
        pltpu.VMEM((2 * SIB, CH2), jnp.int32),
        pltpu.VMEM((CH2, H), jnp.float32),
        pltpu.VMEM((CH2, H), jnp.float32),
        pltpu.VMEM((CH2, H), jnp.float32),
        pltpu.VMEM_SHARED((NPAD, H), jnp.float32),
        pltpu.SemaphoreType.DMA,
        pltpu.SemaphoreType.DMA,
        pltpu.SemaphoreType.DMA,
    ],
)
def _sc_spmm(m_hbm, idx_hbm, p_hbm, ib,
             r0, r1, r2, agg_sh, s0, s1, s2):
    c = lax.axis_index("c")
    s = lax.axis_index("s")
    rows = (r0, r1, r2)
    sems = (s0, s1, s2)
    zero16 = jnp.zeros((16,), jnp.float32)

    def zbody(r, carry):
        for k2 in range(H // 16):
            r0[r, pl.ds(k2 * 16, 16)] = zero16
        return carry

    lax.fori_loop(0, CH2, zbody, 0)
    base = s * RPS
    off = 0
    for size in (112,) * 5 + (RPS - 560,):
        pltpu.sync_copy(r0.at[pl.ds(0, size)],
                        agg_sh.at[pl.ds(base + off, size)])
        off += size
    plsc.subcore_barrier()

    for b in range(NBLKS):
        @pl.when(c == WHO[b])
        def _(b=b):
            pltpu.sync_copy(idx_hbm.at[s, b], ib)
            for k in range(DEPTH):
                pltpu.async_copy(m_hbm.at[ib.at[k]], rows[k], sems[k])

            def gbody(g, carry):
                for k in range(DEPTH):
                    j = g * DEPTH + k
                    pltpu.make_async_copy(m_hbm.at[ib.at[j]],
                                          rows[k], sems[k]).wait()
                    pltpu.sync_copy(rows[k], agg_sh.at[ib.at[SIB + j]],
                                    add=True)

                    @pl.when(g < NGRP - 1)
                    def _(k=k, j=j):
                        pltpu.async_copy(m_hbm.at[ib.at[j + DEPTH]],
                                         rows[k], sems[k])
                return carry

            lax.fori_loop(0, NGRP, gbody, 0)
    plsc.subcore_barrier()
    pltpu.sync_copy(agg_sh.at[pl.ds(base, RPS)],
                    p_hbm.at[c, pl.ds(base, RPS)])


def _tc1_body(x_ref, w_ref, degp_ref, m_ref, dinv_ref):
    deg = degp_ref[0, :, 0:1] + degp_ref[1, :, 0:1] + 1.0
    dinv = lax.rsqrt(deg)
    dinv_ref[...] = dinv
    m_ref[...] = jnp.dot(x_ref[...], w_ref[...],
                         preferred_element_type=jnp.float32) * dinv


_tc1 = pl.pallas_call(
    _tc1_body,
    grid=(NB,),
    in_specs=[
        pl.BlockSpec((RB, H), lambda i: (i, 0)),
        pl.BlockSpec((H, H), lambda i: (0, 0)),
        pl.BlockSpec((NC, RB, H), lambda i: (0, i, 0)),
    ],
    out_specs=[
        pl.BlockSpec((RB, H), lambda i: (i, 0)),
        pl.BlockSpec((RB, 1), lambda i: (i, 0)),
    ],
    out_shape=[
        jax.ShapeDtypeStruct((N, H), jnp.float32),
        jax.ShapeDtypeStruct((N, 1), jnp.float32),
    ],
)


def _tc_mid_body(p_ref, m_ref, dinv_ref, b_ref, w_ref, o_ref):
    dinv = dinv_ref[...]
    a = (p_ref[0] + p_ref[1] + m_ref[...]) * dinv + b_ref[...]
    a = jnp.maximum(a, 0.0)
    o_ref[...] = jnp.dot(a, w_ref[...],
                         preferred_element_type=jnp.float32) * dinv


_tc_mid = pl.pallas_call(
    _tc_mid_body,
    grid=(NB,),
    in_specs=[
        pl.BlockSpec((NC, RB, H), lambda i: (0, i, 0)),
        pl.BlockSpec((RB, H), lambda i: (i, 0)),
        pl.BlockSpec((RB, 1), lambda i: (i, 0)),
        pl.BlockSpec((1, H), lambda i: (0, 0)),
        pl.BlockSpec((H, H), lambda i: (0, 0)),
    ],
    out_specs=pl.BlockSpec((RB, H), lambda i: (i, 0)),
    out_shape=jax.ShapeDtypeStruct((N, H), jnp.float32),
)


def _tc4_body(p_ref, m_ref, dinv_ref, b_ref, batch_ref, wl_ref, bl_ref,
              o_ref, sums_ref, cnt_ref):
    i = pl.program_id(0)

    @pl.when(i == 0)
    def _():
        sums_ref[...] = jnp.zeros_like(sums_ref)
        cnt_ref[...] = jnp.zeros_like(cnt_ref)

    h = (p_ref[0] + p_ref[1] + m_ref[...]) * dinv_ref[...] + b_ref[...]
    onehot = (batch_ref[...] ==
              lax.broadcasted_iota(jnp.int32, (RB, G), 1)).astype(jnp.float32)
    sums_ref[...] += lax.dot_general(
        onehot, h, (((0,), (0,)), ((), ())),
        preferred_element_type=jnp.float32)
    cnt_ref[...] += lax.dot_general(
        onehot, jnp.ones((RB, H), jnp.float32), (((0,), (0,)), ((), ())),
        preferred_element_type=jnp.float32)

    @pl.when(i == NB - 1)
    def _():
        pooled = sums_ref[...] / jnp.maximum(cnt_ref[...], 1.0)
        o_ref[...] = jnp.dot(pooled, wl_ref[...],
                             preferred_element_type=jnp.float32) + bl_ref[...]


_tc4 = pl.pallas_call(
    _tc4_body,
    grid=(NB,),
    in_specs=[
        pl.BlockSpec((NC, RB, H), lambda i: (0, i, 0)),
        pl.BlockSpec((RB, H), lambda i: (i, 0)),
        pl.BlockSpec((RB, 1), lambda i: (i, 0)),
        pl.BlockSpec((1, H), lambda i: (0, 0)),
        pl.BlockSpec((RB, 1), lambda i: (i, 0)),
        pl.BlockSpec((H, DOUT), lambda i: (0, 0)),
        pl.BlockSpec((1, DOUT), lambda i: (0, 0)),
    ],
    out_specs=pl.BlockSpec((G, DOUT), lambda i: (0, 0)),
    out_shape=jax.ShapeDtypeStruct((G, DOUT), jnp.float32),
    scratch_shapes=[
        pltpu.VMEM((G, H), jnp.float32),
        pltpu.VMEM((G, H), jnp.float32),
    ],
)


def kernel(x, edge_index, batch, W1, b1, W2, b2, W3, b3, Wl, bl):
    pad = EPAD - E
    dst_p = jnp.concatenate([edge_index[1], jnp.full((pad,), N, jnp.int32)])
    dst3 = dst_p.reshape(NW, NCH, CHUNK)
    pad2 = EPADS - E
    src_p2 = jnp.concatenate([edge_index[0], jnp.zeros((pad2,), jnp.int32)])
    dst_p2 = jnp.concatenate([edge_index[1], jnp.full((pad2,), N, jnp.int32)])
    src4 = src_p2.reshape(NS, NBLKS, SIB, CH2)
    dst4 = dst_p2.reshape(NS, NBLKS, SIB, CH2)
    idx_comb = jnp.concatenate([src4, dst4], axis=2).reshape(
        NS, NBLKS, 2 * SIB, CH2)
    batch2 = batch.reshape(N, 1)

    degp = _sc_deg(dst3)
    m1, dinv = _tc1(x, W1, degp)
    p1 = _sc_spmm(m1, idx_comb)
    m2 = _tc_mid(p1, m1, dinv, b1.reshape(1, H), W2)
    p2 = _sc_spmm(m2, idx_comb)
    m3 = _tc_mid(p2, m2, dinv, b2.reshape(1, H), W3)
    p3 = _sc_spmm(m3, idx_comb)
    return _tc4(p3, m3, dinv, b3.reshape(1, H), batch2, Wl,
                bl.reshape(1, DOUT))
